# baseline (speedup 1.0000x reference)
import functools

import jax
import jax.numpy as jnp
from jax import lax
from jax.experimental import pallas as pl

K = 16
DILATION = 2
TOPK = K * DILATION
N = 4096
CIN = 512
COUT = 256

R1 = 512
R2 = 128


def _feat_body(x_ref, w_ref, s_ref, t_ref, feat_ref, qn_ref):
    xb = x_ref[0]
    w = w_ref[...]
    feat = lax.dot_general(
        xb.astype(jnp.bfloat16), w.astype(jnp.bfloat16),
        (((0,), (1,)), ((), ())),
        preferred_element_type=jnp.float32)
    feat = feat * s_ref[0] + t_ref[0]
    ns = jnp.sum(feat * feat, axis=1, keepdims=True)
    inv = 1.0 / jnp.maximum(jnp.sqrt(ns), 1e-12)
    feat_ref[0] = feat.astype(jnp.bfloat16)
    qn_ref[0] = (feat * inv).astype(jnp.bfloat16)


def _dist_body(qr_ref, qf_ref, nr_ref, nf_ref, rp_ref, sim_ref, nn_ref):
    d0 = lax.dot_general(
        qr_ref[0], qf_ref[0], (((1,), (1,)), ((), ())),
        preferred_element_type=jnp.float32)
    m = jnp.max(d0, axis=-1, keepdims=True)
    e = jnp.exp(d0 - m)
    sim_ref[0] = e * (1.0 / jnp.sum(e, axis=-1, keepdims=True))

    dn = lax.dot_general(
        nr_ref[0], nf_ref[0], (((1,), (1,)), ((), ())),
        preferred_element_type=jnp.float32)
    vals = dn + rp_ref[0]
    iotaf = lax.broadcasted_iota(jnp.int32, (R2, N), 1).astype(jnp.float32)
    cols = []
    for j in range(TOPK - 1):
        mx = jnp.max(vals, axis=-1, keepdims=True)
        hit = vals == mx
        if j % 2 == 0:
            cols.append(jnp.min(jnp.where(hit, iotaf, jnp.float32(N)),
                                axis=-1, keepdims=True))
        if j < TOPK - 2:
            vals = jnp.where(hit, -jnp.inf, vals)
    nn_ref[0] = jnp.concatenate(cols, axis=1).astype(jnp.int32)


@jax.jit
def kernel(x, relative_pos, W, b, gamma, beta):
    B = x.shape[0]
    x2d = x.reshape(B, CIN, N)
    s = (gamma * (1.0 / jnp.sqrt(jnp.float32(1.0 + 1e-5)))).reshape(1, COUT)
    t = (b * s[0] + beta).reshape(1, COUT)

    feat_bf, qn_bf = pl.pallas_call(
        _feat_body,
        grid=(B, N // R1),
        in_specs=[
            pl.BlockSpec((1, CIN, R1), lambda bi, i: (bi, 0, i)),
            pl.BlockSpec((COUT, CIN), lambda bi, i: (0, 0)),
            pl.BlockSpec((1, COUT), lambda bi, i: (0, 0)),
            pl.BlockSpec((1, COUT), lambda bi, i: (0, 0)),
        ],
        out_specs=[
            pl.BlockSpec((1, R1, COUT), lambda bi, i: (bi, i, 0)),
            pl.BlockSpec((1, R1, COUT), lambda bi, i: (bi, i, 0)),
        ],
        out_shape=[
            jax.ShapeDtypeStruct((B, N, COUT), jnp.bfloat16),
            jax.ShapeDtypeStruct((B, N, COUT), jnp.bfloat16),
        ],
    )(x2d, W, s, t)

    sim, nn_idx = pl.pallas_call(
        _dist_body,
        grid=(B, N // R2),
        in_specs=[
            pl.BlockSpec((1, R2, COUT), lambda bi, i: (bi, i, 0)),
            pl.BlockSpec((1, N, COUT), lambda bi, i: (bi, 0, 0)),
            pl.BlockSpec((1, R2, COUT), lambda bi, i: (bi, i, 0)),
            pl.BlockSpec((1, N, COUT), lambda bi, i: (bi, 0, 0)),
            pl.BlockSpec((1, R2, N), lambda bi, i: (bi, i, 0)),
        ],
        out_specs=[
            pl.BlockSpec((1, R2, N), lambda bi, i: (bi, i, 0)),
            pl.BlockSpec((1, R2, K), lambda bi, i: (bi, i, 0)),
        ],
        out_shape=[
            jax.ShapeDtypeStruct((B, N, N), jnp.float32),
            jax.ShapeDtypeStruct((B, N, K), jnp.int32),
        ],
    )(feat_bf, feat_bf, qn_bf, qn_bf, relative_pos)

    center = jnp.broadcast_to(
        jnp.arange(N, dtype=jnp.int32)[None, :, None], (B, N, K))
    edge_index = jnp.stack((nn_idx, center), axis=0)
    return (edge_index, sim)


# speedup vs baseline: 1.0373x; 1.0373x over previous
import functools

import jax
import jax.numpy as jnp
from jax import lax
from jax.experimental import pallas as pl

K = 16
DILATION = 2
TOPK = K * DILATION
N = 4096
CIN = 512
COUT = 256

R1 = 512
R2 = 256


def _feat_body(x_ref, w_ref, s_ref, t_ref, feat_ref, qn_ref):
    xb = x_ref[0]
    w = w_ref[...]
    feat = lax.dot_general(
        xb.astype(jnp.bfloat16), w.astype(jnp.bfloat16),
        (((0,), (1,)), ((), ())),
        preferred_element_type=jnp.float32)
    feat = feat * s_ref[0] + t_ref[0]
    ns = jnp.sum(feat * feat, axis=1, keepdims=True)
    inv = 1.0 / jnp.maximum(jnp.sqrt(ns), 1e-12)
    feat_ref[0] = feat.astype(jnp.bfloat16)
    qn_ref[0] = (feat * inv).astype(jnp.bfloat16)


def _dist_body(qr_ref, qf_ref, nr_ref, nf_ref, rp_ref, sim_ref, nn_ref):
    d0 = lax.dot_general(
        qr_ref[0], qf_ref[0], (((1,), (1,)), ((), ())),
        preferred_element_type=jnp.float32)
    m = jnp.max(d0, axis=-1, keepdims=True)
    e = jnp.exp(d0 - m)
    sim_ref[0] = e * (1.0 / jnp.sum(e, axis=-1, keepdims=True))

    dn = lax.dot_general(
        nr_ref[0], nf_ref[0], (((1,), (1,)), ((), ())),
        preferred_element_type=jnp.float32)
    vals = dn + rp_ref[0]
    iota = lax.broadcasted_iota(jnp.int32, (R2, N), 1)
    cols = []
    for j in range(TOPK - 1):
        mx = jnp.max(vals, axis=-1, keepdims=True)
        hit = vals == mx
        if j % 2 == 0:
            cols.append(jnp.min(jnp.where(hit, iota, N), axis=-1,
                                keepdims=True))
        if j < TOPK - 2:
            vals = jnp.where(hit, -jnp.inf, vals)
    nn_ref[0] = jnp.concatenate(cols, axis=1)


@jax.jit
def kernel(x, relative_pos, W, b, gamma, beta):
    B = x.shape[0]
    x2d = x.reshape(B, CIN, N)
    s = (gamma * (1.0 / jnp.sqrt(jnp.float32(1.0 + 1e-5)))).reshape(1, COUT)
    t = (b * s[0] + beta).reshape(1, COUT)

    feat_bf, qn_bf = pl.pallas_call(
        _feat_body,
        grid=(B, N // R1),
        in_specs=[
            pl.BlockSpec((1, CIN, R1), lambda bi, i: (bi, 0, i)),
            pl.BlockSpec((COUT, CIN), lambda bi, i: (0, 0)),
            pl.BlockSpec((1, COUT), lambda bi, i: (0, 0)),
            pl.BlockSpec((1, COUT), lambda bi, i: (0, 0)),
        ],
        out_specs=[
            pl.BlockSpec((1, R1, COUT), lambda bi, i: (bi, i, 0)),
            pl.BlockSpec((1, R1, COUT), lambda bi, i: (bi, i, 0)),
        ],
        out_shape=[
            jax.ShapeDtypeStruct((B, N, COUT), jnp.bfloat16),
            jax.ShapeDtypeStruct((B, N, COUT), jnp.bfloat16),
        ],
    )(x2d, W, s, t)

    sim, nn_idx = pl.pallas_call(
        _dist_body,
        grid=(B, N // R2),
        in_specs=[
            pl.BlockSpec((1, R2, COUT), lambda bi, i: (bi, i, 0)),
            pl.BlockSpec((1, N, COUT), lambda bi, i: (bi, 0, 0)),
            pl.BlockSpec((1, R2, COUT), lambda bi, i: (bi, i, 0)),
            pl.BlockSpec((1, N, COUT), lambda bi, i: (bi, 0, 0)),
            pl.BlockSpec((1, R2, N), lambda bi, i: (bi, i, 0)),
        ],
        out_specs=[
            pl.BlockSpec((1, R2, N), lambda bi, i: (bi, i, 0)),
            pl.BlockSpec((1, R2, K), lambda bi, i: (bi, i, 0)),
        ],
        out_shape=[
            jax.ShapeDtypeStruct((B, N, N), jnp.float32),
            jax.ShapeDtypeStruct((B, N, K), jnp.int32),
        ],
    )(feat_bf, feat_bf, qn_bf, qn_bf, relative_pos)

    center = jnp.broadcast_to(
        jnp.arange(N, dtype=jnp.int32)[None, :, None], (B, N, K))
    edge_index = jnp.stack((nn_idx, center), axis=0)
    return (edge_index, sim)
